# baseline (speedup 1.0000x reference)
import functools

import jax
import jax.numpy as jnp
from jax import lax
from jax.experimental import pallas as pl
from jax.experimental.pallas import tpu as pltpu
from jax.experimental.pallas import tpu_sc as plsc

OUT_SIZE = 64
PAD_W = 128
BATCH = 16384
HIST = 50
HIST_P = 56
ROWS = BATCH * HIST_P
ROWS2 = 2 * ROWS

NC, NS = 2, 16
NW = NC * NS
ROWS2_W = ROWS2 // NW
STREAM = 128
NCHUNK = ROWS2_W // STREAM
NBUF = 4


def _gather_body(idx_hbm, table_hbm, out_hbm, idx_v, rows_v, *sems):
    gsems, ssems = sems[:NBUF], sems[NBUF:]
    wid = lax.axis_index("s") * NC + lax.axis_index("c")
    pltpu.sync_copy(idx_hbm.at[wid], idx_v)
    out_base = wid * ROWS2_W

    def issue_gather(j, b):
        pltpu.async_copy(table_hbm.at[idx_v.at[j]], rows_v.at[b], gsems[b])

    def wait_gather(j, b):
        pltpu.make_async_copy(
            table_hbm.at[idx_v.at[j]], rows_v.at[b], gsems[b]).wait()

    def store_descr(j, b):
        return (rows_v.at[b],
                out_hbm.at[pl.ds(out_base + j * STREAM, STREAM)], ssems[b])

    for b in range(NBUF - 1):
        issue_gather(b, b)

    def body(g, carry):
        for b in range(NBUF):
            j = g * NBUF + b
            bp = (b + NBUF - 1) % NBUF
            wait_gather(j, b)
            pltpu.async_copy(*store_descr(j, b))

            @pl.when(j >= 1)
            def _():
                pltpu.make_async_copy(*store_descr(j - 1, bp)).wait()

            @pl.when(j + NBUF - 1 < NCHUNK)
            def _():
                issue_gather(j + NBUF - 1, bp)
        return carry

    lax.fori_loop(0, NCHUNK // NBUF, body, 0)
    pltpu.make_async_copy(*store_descr(NCHUNK - 1, (NCHUNK - 1) % NBUF)).wait()


@functools.partial(jax.jit, static_argnums=())
def _run(idx, table):
    k = pl.kernel(
        _gather_body,
        out_type=jax.ShapeDtypeStruct((ROWS2, OUT_SIZE), jnp.float32),
        mesh=plsc.VectorSubcoreMesh(core_axis_name="c", subcore_axis_name="s"),
        scratch_types=[
            pltpu.VMEM((NCHUNK, STREAM), jnp.int32),
            pltpu.VMEM((NBUF, STREAM, OUT_SIZE), jnp.float32),
        ] + [pltpu.SemaphoreType.DMA] * (2 * NBUF),
        compiler_params=pltpu.CompilerParams(use_tc_tiling_on_sc=False),
    )
    return k(idx, table)


def kernel(inputs, embeddings):
    ids = jnp.pad(inputs.astype(jnp.int32), ((0, 0), (0, HIST_P - HIST)))
    idx2 = jnp.stack([ids, jnp.zeros_like(ids)], axis=-1)
    idx2 = idx2.reshape(NW, NCHUNK, STREAM)
    out = _run(idx2, embeddings)
    return out.reshape(BATCH, HIST_P, PAD_W)[:, :HIST, :OUT_SIZE]


# speedup vs baseline: 3.3301x; 3.3301x over previous
import functools

import jax
import jax.numpy as jnp
from jax import lax
from jax.experimental import pallas as pl
from jax.experimental.pallas import tpu as pltpu
from jax.experimental.pallas import tpu_sc as plsc

OUT_SIZE = 64
PAD_W = 128
BATCH = 16384
HIST = 50
HIST_P = 56
ROWS = BATCH * HIST_P
ROWS2 = 2 * ROWS

NC, NS = 2, 16
NW = NC * NS
ROWS2_W = ROWS2 // NW
STREAM = 128
NCHUNK = ROWS2_W // STREAM
NBUF = 4


def _gather_body(idx_hbm, table_hbm, out_hbm, idx_v, rows_v, *sems):
    gsems, ssems = sems[:NBUF], sems[NBUF:]
    wid = lax.axis_index("s") * NC + lax.axis_index("c")
    pltpu.sync_copy(idx_hbm.at[wid], idx_v)
    out_base = wid * ROWS2_W

    def issue_gather(j, b):
        pltpu.async_copy(table_hbm.at[idx_v.at[j]], rows_v.at[b], gsems[b])

    def wait_gather(j, b):
        pltpu.make_async_copy(
            table_hbm.at[idx_v.at[j]], rows_v.at[b], gsems[b]).wait()

    def store_descr(j, b):
        return (rows_v.at[b],
                out_hbm.at[pl.ds(out_base + j * STREAM, STREAM)], ssems[b])

    for b in range(NBUF - 1):
        issue_gather(b, b)

    def body(g, carry):
        for b in range(NBUF):
            j = g * NBUF + b
            bp = (b + NBUF - 1) % NBUF
            wait_gather(j, b)
            pltpu.async_copy(*store_descr(j, b))

            @pl.when(j >= 1)
            def _():
                pltpu.make_async_copy(*store_descr(j - 1, bp)).wait()

            @pl.when(j + NBUF - 1 < NCHUNK)
            def _():
                issue_gather(j + NBUF - 1, bp)
        return carry

    lax.fori_loop(0, NCHUNK // NBUF, body, 0)
    pltpu.make_async_copy(*store_descr(NCHUNK - 1, (NCHUNK - 1) % NBUF)).wait()


@functools.partial(jax.jit, static_argnums=())
def _run(idx, table):
    k = pl.kernel(
        _gather_body,
        out_type=jax.ShapeDtypeStruct((ROWS2, OUT_SIZE), jnp.float32),
        mesh=plsc.VectorSubcoreMesh(core_axis_name="c", subcore_axis_name="s"),
        scratch_types=[
            pltpu.VMEM((NCHUNK, STREAM), jnp.int32),
            pltpu.VMEM((NBUF, STREAM, OUT_SIZE), jnp.float32),
        ] + [pltpu.SemaphoreType.DMA] * (2 * NBUF),
        compiler_params=pltpu.CompilerParams(use_tc_tiling_on_sc=False),
    )
    return k(idx, table)


def kernel(inputs, embeddings):
    ids = jnp.pad(inputs.astype(jnp.int32), ((0, 0), (0, HIST_P - HIST)))
    idx2 = jnp.stack([ids, ids], axis=-1)
    idx2 = idx2.reshape(NW, NCHUNK, STREAM)
    out = _run(idx2, embeddings)
    return out.reshape(BATCH, HIST_P, PAD_W)[:, :HIST, :OUT_SIZE]


# speedup vs baseline: 9.7751x; 2.9354x over previous
import functools

import jax
import jax.numpy as jnp
from jax import lax
from jax.experimental import pallas as pl
from jax.experimental.pallas import tpu as pltpu
from jax.experimental.pallas import tpu_sc as plsc

OUT_SIZE = 64
PAD_W = 128
BATCH = 16384
HIST = 50
HIST_P = 56
ROWS = BATCH * HIST_P
ROWS2 = 2 * ROWS

NC, NS = 2, 16
NW = NC * NS
ROWS2_W = ROWS2 // NW
STREAM = 128
NCHUNK = ROWS2_W // STREAM
NBUF = 4


def _gather_body(idx_hbm, table_hbm, out_hbm, idx_v, rows_v, *sems):
    gsems, ssems = sems[:NBUF], sems[NBUF:]
    wid = lax.axis_index("s") * NC + lax.axis_index("c")
    pltpu.sync_copy(idx_hbm.at[wid], idx_v)
    out_base = wid * ROWS2_W

    def issue_gather(j, b):
        pltpu.async_copy(table_hbm.at[idx_v.at[j]], rows_v.at[b], gsems[b])

    def wait_gather(j, b):
        pltpu.make_async_copy(
            table_hbm.at[idx_v.at[j]], rows_v.at[b], gsems[b]).wait()

    def store_descr(j, b):
        return (rows_v.at[b],
                out_hbm.at[pl.ds(out_base + j * STREAM, STREAM)], ssems[b])

    for b in range(NBUF - 1):
        issue_gather(b, b)

    def body(g, carry):
        for b in range(NBUF):
            j = g * NBUF + b
            bp = (b + NBUF - 1) % NBUF
            wait_gather(j, b)
            pltpu.async_copy(*store_descr(j, b))

            @pl.when(j >= 1)
            def _():
                pltpu.make_async_copy(*store_descr(j - 1, bp)).wait()

            @pl.when(j + NBUF - 1 < NCHUNK)
            def _():
                issue_gather(j + NBUF - 1, bp)
        return carry

    lax.fori_loop(0, NCHUNK // NBUF, body, 0)
    pltpu.make_async_copy(*store_descr(NCHUNK - 1, (NCHUNK - 1) % NBUF)).wait()


@functools.partial(jax.jit, static_argnums=())
def _run(idx, table):
    k = pl.kernel(
        _gather_body,
        out_type=jax.ShapeDtypeStruct((ROWS2, OUT_SIZE), jnp.float32),
        mesh=plsc.VectorSubcoreMesh(core_axis_name="c", subcore_axis_name="s"),
        scratch_types=[
            pltpu.VMEM((NCHUNK, STREAM), jnp.int32),
            pltpu.VMEM((NBUF, STREAM, OUT_SIZE), jnp.float32),
        ] + [pltpu.SemaphoreType.DMA] * (2 * NBUF),
        compiler_params=pltpu.CompilerParams(use_tc_tiling_on_sc=False),
    )
    return k(idx, table)


def kernel(inputs, embeddings):
    ids = jnp.pad(inputs.astype(jnp.int32), ((0, 0), (0, HIST_P - HIST)),
                  mode='edge')
    idx2 = jnp.stack([ids, ids], axis=-1)
    idx2 = idx2.reshape(NW, NCHUNK, STREAM)
    out = _run(idx2, embeddings)
    return out.reshape(BATCH, HIST_P, PAD_W)[:, :HIST, :OUT_SIZE]


# speedup vs baseline: 9.9614x; 1.0191x over previous
import functools

import jax
import jax.numpy as jnp
from jax import lax
from jax.experimental import pallas as pl
from jax.experimental.pallas import tpu as pltpu
from jax.experimental.pallas import tpu_sc as plsc

OUT_SIZE = 64
PAD_W = 128
BATCH = 16384
HIST = 50
HIST_P = 56
ROWS = BATCH * HIST_P
ROWS2 = 2 * ROWS

NC, NS = 2, 16
NW = NC * NS
ROWS2_W = ROWS2 // NW
STREAM = 128
NCHUNK = ROWS2_W // STREAM
NBUF = 4


def _gather_body(idx_hbm, table_hbm, out_hbm, idx_v, rows_v, *sems):
    gsems, ssems = sems[:NBUF], sems[NBUF:]
    wid = lax.axis_index("s") * NC + lax.axis_index("c")
    pltpu.sync_copy(idx_hbm.at[wid], idx_v)
    out_base = wid * ROWS2_W

    def issue_gather(j, b):
        pltpu.async_copy(table_hbm.at[idx_v.at[j]], rows_v.at[b], gsems[b])

    def wait_gather(j, b):
        pltpu.make_async_copy(
            table_hbm.at[idx_v.at[j]], rows_v.at[b], gsems[b]).wait()

    def store_descr(j, b):
        return (rows_v.at[b],
                out_hbm.at[pl.ds(out_base + j * STREAM, STREAM)], ssems[b])

    for b in range(NBUF - 1):
        issue_gather(b, b)

    def body(g, carry):
        for b in range(NBUF):
            j = g * NBUF + b
            bp = (b + NBUF - 1) % NBUF
            wait_gather(j, b)
            pltpu.async_copy(*store_descr(j, b))

            @pl.when(j >= 1)
            def _():
                pltpu.make_async_copy(*store_descr(j - 1, bp)).wait()

            @pl.when(j + NBUF - 1 < NCHUNK)
            def _():
                issue_gather(j + NBUF - 1, bp)
        return carry

    lax.fori_loop(0, NCHUNK // NBUF, body, 0)
    pltpu.make_async_copy(*store_descr(NCHUNK - 1, (NCHUNK - 1) % NBUF)).wait()


@functools.partial(jax.jit, static_argnums=())
def _run(idx, table):
    k = pl.kernel(
        _gather_body,
        out_type=jax.ShapeDtypeStruct((ROWS2, OUT_SIZE), jnp.float32),
        mesh=plsc.VectorSubcoreMesh(core_axis_name="c", subcore_axis_name="s"),
        scratch_types=[
            pltpu.VMEM((NCHUNK, STREAM), jnp.int32),
            pltpu.VMEM((NBUF, STREAM, OUT_SIZE), jnp.float32),
        ] + [pltpu.SemaphoreType.DMA] * (2 * NBUF),
        compiler_params=pltpu.CompilerParams(use_tc_tiling_on_sc=False),
    )
    return k(idx, table)


def kernel(inputs, embeddings):
    ids = jnp.pad(inputs.astype(jnp.int32), ((0, 0), (0, HIST_P - HIST)),
                  mode='edge')
    junk = (ids + 499993) % 1000000
    idx2 = jnp.stack([ids, junk], axis=-1)
    idx2 = idx2.reshape(NW, NCHUNK, STREAM)
    out = _run(idx2, embeddings)
    return out.reshape(BATCH, HIST_P, PAD_W)[:, :HIST, :OUT_SIZE]


# speedup vs baseline: 17.2464x; 1.7313x over previous
import functools

import jax
import jax.numpy as jnp
from jax import lax
from jax.experimental import pallas as pl
from jax.experimental.pallas import tpu as pltpu
from jax.experimental.pallas import tpu_sc as plsc

OUT_SIZE = 64
BATCH = 16384
HIST = 50
ROWS = BATCH * HIST

NC, NS = 2, 16
NW = NC * NS
ROWS_W = ROWS // NW
STREAM = 128
NCHUNK = ROWS_W // STREAM
NBUF = 4


def _gather_body(idx_hbm, table_hbm, out_hbm, idx_v, rows_v, *sems):
    gsems, ssems = sems[:NBUF], sems[NBUF:]
    wid = lax.axis_index("s") * NC + lax.axis_index("c")
    pltpu.sync_copy(idx_hbm.at[wid], idx_v)
    out_base = wid * ROWS_W

    def issue_gather(j, b):
        pltpu.async_copy(table_hbm.at[idx_v.at[j]], rows_v.at[b], gsems[b])

    def wait_gather(j, b):
        pltpu.make_async_copy(
            table_hbm.at[idx_v.at[j]], rows_v.at[b], gsems[b]).wait()

    def store_descr(j, b):
        return (rows_v.at[b],
                out_hbm.at[pl.ds(out_base + j * STREAM, STREAM)], ssems[b])

    for b in range(NBUF - 1):
        issue_gather(b, b)

    def body(g, carry):
        for b in range(NBUF):
            j = g * NBUF + b
            bp = (b + NBUF - 1) % NBUF
            wait_gather(j, b)
            pltpu.async_copy(*store_descr(j, b))

            @pl.when(j >= 1)
            def _():
                pltpu.make_async_copy(*store_descr(j - 1, bp)).wait()

            @pl.when(j + NBUF - 1 < NCHUNK)
            def _():
                issue_gather(j + NBUF - 1, bp)
        return carry

    lax.fori_loop(0, NCHUNK // NBUF, body, 0)
    pltpu.make_async_copy(*store_descr(NCHUNK - 1, (NCHUNK - 1) % NBUF)).wait()


@functools.partial(jax.jit, static_argnums=())
def _run(idx, table):
    k = pl.kernel(
        _gather_body,
        out_type=jax.ShapeDtypeStruct((ROWS, OUT_SIZE), jnp.float32),
        mesh=plsc.VectorSubcoreMesh(core_axis_name="c", subcore_axis_name="s"),
        scratch_types=[
            pltpu.VMEM((NCHUNK, STREAM), jnp.int32),
            pltpu.VMEM((NBUF, STREAM, OUT_SIZE), jnp.float32),
        ] + [pltpu.SemaphoreType.DMA] * (2 * NBUF),
        compiler_params=pltpu.CompilerParams(use_tc_tiling_on_sc=False),
    )
    return k(idx, table)


def kernel(inputs, embeddings):
    idx = inputs.astype(jnp.int32).reshape(NW, NCHUNK, STREAM)
    out = _run(idx, embeddings)
    return out.reshape(BATCH, HIST, OUT_SIZE)
